# baseline (speedup 1.0000x reference)
import functools

import jax
import jax.numpy as jnp
from jax import lax
from jax.experimental import pallas as pl
import jax.experimental.pallas.tpu as pltpu

VW = 32
VD = 32
VH = 32
_V = VW * VD * VH
_NF = 128
_NV = 128
_BS = 2


_NL = _BS * _NF


def _voxelize_kernel(vt_ref, fidx_ref, out_ref, acc_ref, pt_ref):
    f32 = jnp.float32
    vt = vt_ref[...]
    fr = fidx_ref[...]

    iota_v = lax.broadcasted_iota(jnp.int32, (2 * _BS * _NV, _NL), 0)
    g = []
    for k in range(4):
        oh = (iota_v == fr[k:k + 1, :]).astype(f32)
        g.append(lax.dot_general(
            vt, oh, (((1,), (0,)), ((), ())),
            precision=jax.lax.Precision.HIGHEST,
            preferred_element_type=f32))
    x0, y0, z0 = g[0][0:1, :], g[0][1:2, :], g[0][2:3, :]
    x1, y1, z1 = g[1][0:1, :], g[1][1:2, :], g[1][2:3, :]
    x2, y2, z2 = g[2][0:1, :], g[2][1:2, :], g[2][2:3, :]
    x3, y3, z3 = g[3][0:1, :], g[3][1:2, :], g[3][2:3, :]

    t11 = x0 - x3; t12 = x1 - x3; t13 = x2 - x3
    t21 = y0 - y3; t22 = y1 - y3; t23 = y2 - y3
    t31 = z0 - z3; t32 = z1 - z3; t33 = z2 - z3
    ti11 = t22 * t33 - t23 * t32
    ti12 = t13 * t32 - t12 * t33
    ti13 = t12 * t23 - t13 * t22
    ti21 = t23 * t31 - t21 * t33
    ti22 = t11 * t33 - t13 * t31
    ti23 = t13 * t21 - t11 * t23
    ti31 = t21 * t32 - t22 * t31
    ti32 = t12 * t31 - t11 * t32
    ti33 = t11 * t22 - t12 * t21
    t_det = (t11 * (t22 * t33 - t23 * t32)
             - t12 * (t21 * t33 - t23 * t31)
             + t13 * (t21 * t32 - t22 * t31))
    rdet = 1.0 / t_det

    params = jnp.concatenate(
        [ti11, ti12, ti13, ti21, ti22, ti23, ti31, ti32, ti33,
         x3, y3, z3, rdet, jnp.zeros((3, _NL), f32)], axis=0)

    lv = lax.broadcasted_iota(jnp.int32, (1, _V), 1)
    rowi8 = lax.broadcasted_iota(jnp.int32, (8, 1), 0)
    rowi64 = lax.broadcasted_iota(jnp.int32, (64, 1), 0)
    gxf = (2.0 * (lv // (VD * VH)).astype(f32) + 1.0 - VW) * (1.0 / VW)
    gyf = (2.0 * ((lv // VH) % VD).astype(f32) + 1.0 - VD) * (1.0 / VD)
    gzf = (2.0 * (lv % VH).astype(f32) + 1.0 - VH) * (1.0 / VH)
    rsel8 = rowi64 % 8
    rdiv8 = rowi64 // 8
    grid64 = jnp.where(rsel8 == 0, gxf,
                       jnp.where(rsel8 == 1, gyf,
                                 jnp.where(rsel8 == 2, gzf, 0.0)))
    pt_ref[...] = jnp.transpose(params)
    p3 = params[9:12]
    m64 = jnp.where(rsel8 == 0, p3[0:1],
                    jnp.where(rsel8 == 1, p3[1:2],
                              jnp.where(rsel8 == 2, p3[2:3], 0.0)))
    lane = lax.broadcasted_iota(jnp.int32, (1, _NL), 1)
    ci64 = lax.broadcasted_iota(jnp.int32, (8, 64), 1)
    ri8b = lax.broadcasted_iota(jnp.int32, (8, 64), 0)
    selmask = [(ci64 // 8 == ri8b) & (ci64 % 8 == j) for j in range(3)]

    for b in range(_BS):
        acc_ref[...] = jnp.zeros((8, _V), f32)
        base = b * _NF

        def group_body(gi, carry):
            v3pat = jnp.sum(jnp.where(lane == base + 8 * gi + rdiv8, m64, 0.0),
                            axis=1, keepdims=True)
            d64 = grid64 - v3pat
            col8 = pt_ref[pl.ds(base + 8 * gi, 8), :]
            rbs = []
            for i in range(3):
                tc = [col8[:, 3 * i + j:3 * i + j + 1] for j in range(3)]
                rbs.append(jnp.where(selmask[0], tc[0],
                                     jnp.where(selmask[1], tc[1],
                                               jnp.where(selmask[2], tc[2],
                                                         0.0))))
            biga = jnp.concatenate([rbs[0], rbs[1], rbs[2]], axis=0)
            num32 = lax.dot_general(biga, d64, (((1,), (0,)), ((), ())),
                                    preferred_element_type=f32)

            rf8 = col8[:, 12:13]
            l0 = num32[0:8] * rf8
            l1 = num32[8:16] * rf8
            l2 = num32[16:24] * rf8
            l4 = 1.0 - ((l0 + l1) + l2)
            m = jnp.minimum(jnp.minimum(l0, l1), jnp.minimum(l2, l4))
            acc_ref[...] = jnp.where(m >= 0.0, 1.0, acc_ref[...])
            return carry

        lax.fori_loop(0, _NF // 8, group_body, 0)

        a8 = acc_ref[...]
        a4 = jnp.maximum(a8[0:4], a8[4:8])
        a2 = jnp.maximum(a4[0:2], a4[2:4])
        out_ref[b:b + 1, :] = jnp.maximum(a2[0:1], a2[1:2])


@jax.jit
def kernel(vertices, facets):
    f32 = jnp.float32
    bs, nv = vertices.shape[:2]
    vt = jnp.transpose(vertices.reshape(bs * nv, 3))
    vt = jnp.pad(vt, ((0, 5), (0, 2 * _BS * _NV - bs * nv))).astype(f32)
    offs = (jnp.arange(bs, dtype=jnp.int32) * nv)[:, None, None]
    fidx = facets.astype(jnp.int32) + offs
    fidx = jnp.transpose(fidx, (0, 2, 1))
    fidx = jnp.concatenate([fidx[0], fidx[1]], axis=1)
    fidx = jnp.pad(fidx, ((0, 4), (0, 0)))

    out = pl.pallas_call(
        _voxelize_kernel,
        grid=(1,),
        in_specs=[
            pl.BlockSpec((8, 2 * _BS * _NV), lambda i: (0, 0)),
            pl.BlockSpec((8, _NL), lambda i: (0, 0)),
        ],
        out_specs=pl.BlockSpec((_BS, _V), lambda i: (0, 0)),
        out_shape=jax.ShapeDtypeStruct((_BS, _V), f32),
        scratch_shapes=[
            pltpu.VMEM((8, _V), f32),
            pltpu.VMEM((_NL, 16), f32),
        ],
    )(vt, fidx)
    return out.reshape(bs, VW, VD, VH)


# speedup vs baseline: 1.0071x; 1.0071x over previous
import jax
import jax.numpy as jnp
from jax import lax
from jax.experimental import pallas as pl
import jax.experimental.pallas.tpu as pltpu

VW = 32
VD = 32
VH = 32
_V = VW * VD * VH
_NF = 128
_NV = 128
_BS = 2


_NL = _BS * _NF


def _voxelize_kernel(vt_ref, fidx_ref, out_ref, acc_ref):
    f32 = jnp.float32
    vt = vt_ref[...]
    fr = fidx_ref[...]

    iota_v = lax.broadcasted_iota(jnp.int32, (2 * _BS * _NV, _NL), 0)
    g = []
    for k in range(4):
        oh = (iota_v == fr[k:k + 1, :]).astype(f32)
        g.append(lax.dot_general(
            vt, oh, (((1,), (0,)), ((), ())),
            precision=jax.lax.Precision.HIGHEST,
            preferred_element_type=f32))
    x0, y0, z0 = g[0][0:1, :], g[0][1:2, :], g[0][2:3, :]
    x1, y1, z1 = g[1][0:1, :], g[1][1:2, :], g[1][2:3, :]
    x2, y2, z2 = g[2][0:1, :], g[2][1:2, :], g[2][2:3, :]
    x3, y3, z3 = g[3][0:1, :], g[3][1:2, :], g[3][2:3, :]

    t11 = x0 - x3; t12 = x1 - x3; t13 = x2 - x3
    t21 = y0 - y3; t22 = y1 - y3; t23 = y2 - y3
    t31 = z0 - z3; t32 = z1 - z3; t33 = z2 - z3
    ti11 = t22 * t33 - t23 * t32
    ti12 = t13 * t32 - t12 * t33
    ti13 = t12 * t23 - t13 * t22
    ti21 = t23 * t31 - t21 * t33
    ti22 = t11 * t33 - t13 * t31
    ti23 = t13 * t21 - t11 * t23
    ti31 = t21 * t32 - t22 * t31
    ti32 = t12 * t31 - t11 * t32
    ti33 = t11 * t22 - t12 * t21
    t_det = (t11 * (t22 * t33 - t23 * t32)
             - t12 * (t21 * t33 - t23 * t31)
             + t13 * (t21 * t32 - t22 * t31))
    rdet = 1.0 / t_det

    params = jnp.concatenate(
        [ti11, ti12, ti13, ti21, ti22, ti23, ti31, ti32, ti33,
         x3, y3, z3, rdet, jnp.zeros((3, _NL), f32)], axis=0)

    lv = lax.broadcasted_iota(jnp.int32, (1, _V), 1)
    rowi8 = lax.broadcasted_iota(jnp.int32, (8, 1), 0)
    rowi64 = lax.broadcasted_iota(jnp.int32, (64, 1), 0)
    gxf = (2.0 * (lv // (VD * VH)).astype(f32) + 1.0 - VW) * (1.0 / VW)
    gyf = (2.0 * ((lv // VH) % VD).astype(f32) + 1.0 - VD) * (1.0 / VD)
    gzf = (2.0 * (lv % VH).astype(f32) + 1.0 - VH) * (1.0 / VH)
    rsel8 = rowi64 % 8
    rdiv8 = rowi64 // 8
    grid64 = jnp.where(rsel8 == 0, gxf,
                       jnp.where(rsel8 == 1, gyf,
                                 jnp.where(rsel8 == 2, gzf, 0.0)))
    p3 = params[9:12]
    m64 = jnp.where(rsel8 == 0, p3[0:1],
                    jnp.where(rsel8 == 1, p3[1:2],
                              jnp.where(rsel8 == 2, p3[2:3], 0.0)))

    lane = lax.broadcasted_iota(jnp.int32, (1, _NL), 1)
    ci64 = lax.broadcasted_iota(jnp.int32, (8, 64), 1)
    ri8b = lax.broadcasted_iota(jnp.int32, (8, 64), 0)
    selmask = [(ci64 // 8 == ri8b) & (ci64 % 8 == j) for j in range(3)]

    for b in range(_BS):
        acc_ref[...] = jnp.zeros((8, _V), f32)
        base = b * _NF

        def group_body(gi, carry):
            v3pat = jnp.sum(jnp.where(lane == base + 8 * gi + rdiv8, m64, 0.0),
                            axis=1, keepdims=True)
            d64 = grid64 - v3pat
            rbs = []
            for i in range(3):
                tc = [jnp.sum(jnp.where(lane == base + 8 * gi + rowi8,
                                        params[3 * i + j:3 * i + j + 1, :],
                                        0.0),
                              axis=1, keepdims=True) for j in range(3)]
                rbs.append(jnp.where(selmask[0], tc[0],
                                     jnp.where(selmask[1], tc[1],
                                               jnp.where(selmask[2], tc[2],
                                                         0.0))))
            biga = jnp.concatenate([rbs[0], rbs[1], rbs[2]], axis=0)
            num32 = lax.dot_general(biga, d64, (((1,), (0,)), ((), ())),
                                    preferred_element_type=f32)

            rf8 = jnp.sum(jnp.where(lane == base + 8 * gi + rowi8,
                                    params[12:13, :], 0.0),
                          axis=1, keepdims=True)
            l0 = num32[0:8] * rf8
            l1 = num32[8:16] * rf8
            l2 = num32[16:24] * rf8
            l4 = 1.0 - ((l0 + l1) + l2)
            m = jnp.minimum(jnp.minimum(l0, l1), jnp.minimum(l2, l4))
            acc_ref[...] = jnp.where(m >= 0.0, 1.0, acc_ref[...])
            return carry

        lax.fori_loop(0, _NF // 8, group_body, 0)

        a8 = acc_ref[...]
        a4 = jnp.maximum(a8[0:4], a8[4:8])
        a2 = jnp.maximum(a4[0:2], a4[2:4])
        out_ref[b:b + 1, :] = jnp.maximum(a2[0:1], a2[1:2])


@jax.jit
def kernel(vertices, facets):
    f32 = jnp.float32
    bs, nv = vertices.shape[:2]
    vt = jnp.transpose(vertices.reshape(bs * nv, 3))
    vt = jnp.pad(vt, ((0, 5), (0, 2 * _BS * _NV - bs * nv))).astype(f32)
    offs = (jnp.arange(bs, dtype=jnp.int32) * nv)[:, None, None]
    fidx = facets.astype(jnp.int32) + offs
    fidx = jnp.transpose(fidx, (0, 2, 1))
    fidx = jnp.concatenate([fidx[0], fidx[1]], axis=1)
    fidx = jnp.pad(fidx, ((0, 4), (0, 0)))

    out = pl.pallas_call(
        _voxelize_kernel,
        grid=(1,),
        in_specs=[
            pl.BlockSpec((8, 2 * _BS * _NV), lambda i: (0, 0)),
            pl.BlockSpec((8, _NL), lambda i: (0, 0)),
        ],
        out_specs=pl.BlockSpec((_BS, _V), lambda i: (0, 0)),
        out_shape=jax.ShapeDtypeStruct((_BS, _V), f32),
        scratch_shapes=[
            pltpu.VMEM((8, _V), f32),
        ],
    )(vt, fidx)
    return out.reshape(bs, VW, VD, VH)
